# baseline (speedup 1.0000x reference)
import functools

import jax
import jax.numpy as jnp
from jax import lax
from jax.experimental import pallas as pl
from jax.experimental.pallas import tpu as pltpu
from jax.experimental.pallas import tpu_sc as plsc

B = 16384
D = 256
N_CLASSES = 6
N_SUBJECTS = 100
NPAIR = 640
MARGIN = 1.0
EPS = 1e-6
ROWS = 1024
NBLK = B // ROWS
DROWS = 2048
DNBLK = B // DROWS
I_SENT = 2**30
F_BIG = 1e30


def _mine_body(p_ref, idx_ref, valid2_ref):
    qi = lax.broadcasted_iota(jnp.int32, (ROWS, NPAIR), 1)
    jg = lax.broadcasted_iota(jnp.int32, (ROWS, NPAIR), 0)
    f = jnp.full((NPAIR,), I_SENT, jnp.int32)
    s = jnp.full((NPAIR,), I_SENT, jnp.int32)
    for r in range(NBLK):
        pr = p_ref[r, :][:, None]
        m = jnp.where(pr == qi, jg, I_SENT)
        c1 = jnp.min(m, axis=0)
        c2 = jnp.min(jnp.where(m == c1[None, :], I_SENT, m), axis=0)
        c1 = jnp.where(c1 < I_SENT, c1 + r * ROWS, I_SENT)
        c2 = jnp.where(c2 < I_SENT, c2 + r * ROWS, I_SENT)
        s = jnp.minimum(jnp.maximum(f, c1), jnp.minimum(s, c2))
        f = jnp.minimum(f, c1)
    idx_ref[0:NPAIR] = jnp.minimum(f, B - 1)
    idx_ref[NPAIR:2 * NPAIR] = jnp.minimum(s, B - 1)
    valid2_ref[...] = (s < I_SENT).astype(jnp.int32)


def _mine(p2d):
    return pl.pallas_call(
        _mine_body,
        out_shape=(
            jax.ShapeDtypeStruct((2 * NPAIR,), jnp.int32),
            jax.ShapeDtypeStruct((NPAIR,), jnp.int32),
        ),
    )(p2d)


_ROWS_PER_W = (2 * NPAIR) // 32
NPAD = 656
SAMP_PER_SC = B // 16
SENTP = 2**20


@functools.cache
def _make_sc_mine_gather():
    mesh = plsc.VectorSubcoreMesh(core_axis_name="c", subcore_axis_name="s")

    @functools.partial(
        pl.kernel,
        mesh=mesh,
        compiler_params=pltpu.CompilerParams(needs_layout_passes=False),
        out_type=(
            jax.ShapeDtypeStruct((2 * NPAIR, D), jnp.float32),
            jax.ShapeDtypeStruct((NPAIR,), jnp.int32),
        ),
        scratch_types=[
            pltpu.VMEM((SAMP_PER_SC,), jnp.int32),
            pltpu.VMEM((16 * NPAD,), jnp.int32),
            pltpu.VMEM((16 * NPAD,), jnp.int32),
            pltpu.VMEM((NPAD,), jnp.int32),
            pltpu.VMEM((NPAD,), jnp.int32),
            pltpu.VMEM((2 * 16 * NPAD,), jnp.int32),
            pltpu.VMEM((48,), jnp.int32),
            pltpu.VMEM((48,), jnp.int32),
            pltpu.VMEM((48, D), jnp.float32),
            pltpu.VMEM_SHARED((2 * 16 * NPAD,), jnp.int32),
            pltpu.SemaphoreType.DMA,
        ],
    )
    def _g(emb_hbm, p_hbm, rows_hbm, valid2_hbm, pchunk, loc1, loc2, red1,
           red2, mrg, idx_v, valid_v, rows_v, sh, sem):
        cid = lax.axis_index("c")
        sid = lax.axis_index("s")
        lane = lax.broadcasted_iota(jnp.int32, (16,), 0)
        sent16 = jnp.full((16,), I_SENT, jnp.int32)

        nchunk = NPAD // 16

        def init_body(i, c):
            loc1[pl.ds(i * 16, 16)] = sent16
            loc2[pl.ds(i * 16, 16)] = sent16
            return c
        lax.fori_loop(0, 16 * nchunk, init_body, 0)

        pltpu.sync_copy(p_hbm.at[pl.ds(sid * SAMP_PER_SC, SAMP_PER_SC)],
                        pchunk)

        lane_off = lane * NPAD

        def scan_body(k, c):
            pv = pchunk[pl.ds(k * 16, 16)]
            jv = sid * SAMP_PER_SC + k * 16 + lane
            fidx = lane_off + pv
            cur1 = plsc.load_gather(loc1, [fidx])
            cur2 = plsc.load_gather(loc2, [fidx])
            plsc.store_scatter(loc1, [fidx], jnp.minimum(cur1, jv))
            plsc.store_scatter(loc2, [fidx],
                               jnp.minimum(cur2, jnp.maximum(cur1, jv)))
            return c
        lax.fori_loop(0, SAMP_PER_SC // 16, scan_body, 0)

        def red_body(i, c):
            off = i * 16
            f = sent16
            s = sent16
            for l in range(16):
                c1 = loc1[pl.ds(l * NPAD + off, 16)]
                c2 = loc2[pl.ds(l * NPAD + off, 16)]
                s = jnp.minimum(jnp.maximum(f, c1), jnp.minimum(s, c2))
                f = jnp.minimum(f, c1)
            red1[pl.ds(off, 16)] = f
            red2[pl.ds(off, 16)] = s
            return c
        lax.fori_loop(0, nchunk, red_body, 0)

        pltpu.sync_copy(red1, sh.at[pl.ds(sid * NPAD, NPAD)])
        pltpu.sync_copy(red2, sh.at[pl.ds((16 + sid) * NPAD, NPAD)])
        plsc.subcore_barrier()
        pltpu.sync_copy(sh, mrg)

        pbase = sid * _ROWS_PER_W
        for k in range(3):
            off = pbase + k * 16
            f = sent16
            s = sent16
            for t in range(16):
                c1 = mrg[pl.ds(t * NPAD + off, 16)]
                c2 = mrg[pl.ds((16 + t) * NPAD + off, 16)]
                s = jnp.minimum(jnp.maximum(f, c1), jnp.minimum(s, c2))
                f = jnp.minimum(f, c1)
            my = jnp.where(cid == 0, f, s)
            idx_v[pl.ds(k * 16, 16)] = jnp.minimum(my, B - 1)
            valid_v[pl.ds(k * 16, 16)] = (s < I_SENT).astype(jnp.int32)

        pltpu.async_copy(emb_hbm.at[idx_v], rows_v, sem).wait()
        rowbase = jnp.where(cid == 0, pbase, NPAIR + pbase)
        pltpu.sync_copy(rows_v.at[pl.ds(0, _ROWS_PER_W)],
                        rows_hbm.at[pl.ds(rowbase, _ROWS_PER_W)])

        @pl.when(cid == 0)
        def _():
            pltpu.sync_copy(valid_v.at[pl.ds(0, _ROWS_PER_W)],
                            valid2_hbm.at[pl.ds(pbase, _ROWS_PER_W)])

    return _g


def _sc_mine_gather(emb, p):
    return _make_sc_mine_gather()(emb, p)


def _dist_body(emb_ref, p_ref, a_ref, pos_ref, valid2_ref, loss_ref,
               minacc, qi16, qs8):
    step = pl.program_id(0)

    @pl.when(step == 0)
    def _():
        minacc[...] = jnp.full((16, NPAIR), F_BIG, jnp.bfloat16)
        q = lax.broadcasted_iota(jnp.int32, (DROWS, NPAIR), 1)
        qi16[...] = q.astype(jnp.int16)
        qs8[...] = (q // N_CLASSES).astype(jnp.int8)

    E = emb_ref[...]
    Eb = E.astype(jnp.bfloat16)
    Ab = a_ref[...].astype(jnp.bfloat16)
    G = lax.dot_general(Eb, Ab, (((1,), (1,)), ((), ())),
                        preferred_element_type=jnp.float32)
    en = jnp.sum(E * E, axis=1).astype(jnp.bfloat16)
    val = en[:, None] - 2.0 * G.astype(jnp.bfloat16)
    pr = p_ref[0, 0, :]
    pr16 = pr.astype(jnp.int16)[:, None]
    ps8 = (pr // N_CLASSES).astype(jnp.int8)[:, None]
    mask = (ps8 == qs8[...]) & (pr16 != qi16[...])
    masked = jnp.where(mask, val, jnp.bfloat16(F_BIG))
    mstep = jnp.min(masked.reshape(DROWS // 16, 16, NPAIR), axis=0)
    minacc[...] = jnp.minimum(minacc[...], mstep)

    @pl.when(step == DNBLK - 1)
    def _():
        A = a_ref[...]
        P = pos_ref[...]
        an2 = jnp.sum(A * A, axis=1)
        mn = jnp.min(minacc[...], axis=0).astype(jnp.float32)
        d_an = jnp.sqrt(jnp.maximum(mn + an2, 0.0))
        dif = A - P + EPS
        d_ap = jnp.sqrt(jnp.sum(dif * dif, axis=1))
        valid = (valid2_ref[...] > 0) & (mn < 1e29)
        term = jnp.where(valid, jnp.maximum(d_ap - d_an + MARGIN, 0.0), 0.0)
        total = jnp.sum(term)
        count = jnp.sum(valid.astype(jnp.float32))
        loss = jnp.where(count > 0.0, total / jnp.maximum(count, 1.0),
                         jnp.float32(0.0))
        loss_ref[...] = jnp.broadcast_to(loss, (1, 1))


def _dist(emb, p3, a_rows, p_rows, valid2):
    return pl.pallas_call(
        _dist_body,
        grid=(DNBLK,),
        in_specs=[
            pl.BlockSpec((DROWS, D), lambda s: (s, 0)),
            pl.BlockSpec((1, 1, DROWS), lambda s: (s, 0, 0)),
            pl.BlockSpec((NPAIR, D), lambda s: (0, 0)),
            pl.BlockSpec((NPAIR, D), lambda s: (1, 0)),
            pl.BlockSpec((NPAIR,), lambda s: (0,)),
        ],
        out_specs=pl.BlockSpec((1, 1), lambda s: (0, 0)),
        out_shape=jax.ShapeDtypeStruct((1, 1), jnp.float32),
        scratch_shapes=[pltpu.VMEM((16, NPAIR), jnp.bfloat16),
                        pltpu.VMEM((DROWS, NPAIR), jnp.int16),
                        pltpu.VMEM((DROWS, NPAIR), jnp.int8)],
    )(emb, p3, a_rows, p_rows, valid2)


def kernel(emb, labels, sbj):
    p = sbj * N_CLASSES + labels
    rows, valid2 = _sc_mine_gather(emb, p)
    loss = _dist(emb, p.reshape(DNBLK, 1, DROWS), rows, rows, valid2)
    return loss.reshape(())


# speedup vs baseline: 1.0424x; 1.0424x over previous
import functools

import jax
import jax.numpy as jnp
from jax import lax
from jax.experimental import pallas as pl
from jax.experimental.pallas import tpu as pltpu
from jax.experimental.pallas import tpu_sc as plsc

B = 16384
D = 256
N_CLASSES = 6
N_SUBJECTS = 100
NPAIR = 640
MARGIN = 1.0
EPS = 1e-6
ROWS = 1024
NBLK = B // ROWS
DROWS = 2048
DNBLK = B // DROWS
I_SENT = 2**30
F_BIG = 1e30


def _mine_body(p_ref, idx_ref, valid2_ref):
    qi = lax.broadcasted_iota(jnp.int32, (ROWS, NPAIR), 1)
    jg = lax.broadcasted_iota(jnp.int32, (ROWS, NPAIR), 0)
    f = jnp.full((NPAIR,), I_SENT, jnp.int32)
    s = jnp.full((NPAIR,), I_SENT, jnp.int32)
    for r in range(NBLK):
        pr = p_ref[r, :][:, None]
        m = jnp.where(pr == qi, jg, I_SENT)
        c1 = jnp.min(m, axis=0)
        c2 = jnp.min(jnp.where(m == c1[None, :], I_SENT, m), axis=0)
        c1 = jnp.where(c1 < I_SENT, c1 + r * ROWS, I_SENT)
        c2 = jnp.where(c2 < I_SENT, c2 + r * ROWS, I_SENT)
        s = jnp.minimum(jnp.maximum(f, c1), jnp.minimum(s, c2))
        f = jnp.minimum(f, c1)
    idx_ref[0:NPAIR] = jnp.minimum(f, B - 1)
    idx_ref[NPAIR:2 * NPAIR] = jnp.minimum(s, B - 1)
    valid2_ref[...] = (s < I_SENT).astype(jnp.int32)


def _mine(p2d):
    return pl.pallas_call(
        _mine_body,
        out_shape=(
            jax.ShapeDtypeStruct((2 * NPAIR,), jnp.int32),
            jax.ShapeDtypeStruct((NPAIR,), jnp.int32),
        ),
    )(p2d)


_ROWS_PER_W = (2 * NPAIR) // 32
NPAD = 656
SAMP_PER_SC = B // 16
SENTP = 2**20


@functools.cache
def _make_sc_mine_gather():
    mesh = plsc.VectorSubcoreMesh(core_axis_name="c", subcore_axis_name="s")

    @functools.partial(
        pl.kernel,
        mesh=mesh,
        compiler_params=pltpu.CompilerParams(needs_layout_passes=False),
        out_type=(
            jax.ShapeDtypeStruct((2 * NPAIR, D), jnp.float32),
            jax.ShapeDtypeStruct((NPAIR,), jnp.int32),
        ),
        scratch_types=[
            pltpu.VMEM((SAMP_PER_SC,), jnp.int32),
            pltpu.VMEM((16 * NPAD,), jnp.int32),
            pltpu.VMEM((16 * NPAD,), jnp.int32),
            pltpu.VMEM((NPAD,), jnp.int32),
            pltpu.VMEM((NPAD,), jnp.int32),
            pltpu.VMEM((2 * 16 * NPAD,), jnp.int32),
            pltpu.VMEM((48,), jnp.int32),
            pltpu.VMEM((48,), jnp.int32),
            pltpu.VMEM((48, D), jnp.float32),
            pltpu.VMEM_SHARED((2 * 16 * NPAD,), jnp.int32),
            pltpu.SemaphoreType.DMA,
        ],
    )
    def _g(emb_hbm, p_hbm, rows_hbm, valid2_hbm, pchunk, loc1, loc2, red1,
           red2, mrg, idx_v, valid_v, rows_v, sh, sem):
        cid = lax.axis_index("c")
        sid = lax.axis_index("s")
        lane = lax.broadcasted_iota(jnp.int32, (16,), 0)
        sent16 = jnp.full((16,), I_SENT, jnp.int32)

        nchunk = NPAD // 16

        def init_body(i, c):
            for u in range(8):
                loc1[pl.ds((i * 8 + u) * 16, 16)] = sent16
                loc2[pl.ds((i * 8 + u) * 16, 16)] = sent16
            return c
        lax.fori_loop(0, 16 * nchunk // 8, init_body, 0)

        pltpu.sync_copy(p_hbm.at[pl.ds(sid * SAMP_PER_SC, SAMP_PER_SC)],
                        pchunk)

        lane_off = lane * NPAD

        def scan_body(k, c):
            for u in range(4):
                kk = k * 4 + u
                pv = pchunk[pl.ds(kk * 16, 16)]
                jv = sid * SAMP_PER_SC + kk * 16 + lane
                fidx = lane_off + pv
                cur1 = plsc.load_gather(loc1, [fidx])
                cur2 = plsc.load_gather(loc2, [fidx])
                plsc.store_scatter(loc1, [fidx], jnp.minimum(cur1, jv))
                plsc.store_scatter(loc2, [fidx],
                                   jnp.minimum(cur2, jnp.maximum(cur1, jv)))
            return c
        lax.fori_loop(0, SAMP_PER_SC // 64, scan_body, 0)

        def red_body(i, c):
            off = i * 16
            f = sent16
            s = sent16
            for l in range(16):
                c1 = loc1[pl.ds(l * NPAD + off, 16)]
                c2 = loc2[pl.ds(l * NPAD + off, 16)]
                s = jnp.minimum(jnp.maximum(f, c1), jnp.minimum(s, c2))
                f = jnp.minimum(f, c1)
            red1[pl.ds(off, 16)] = f
            red2[pl.ds(off, 16)] = s
            return c
        lax.fori_loop(0, nchunk, red_body, 0)

        pltpu.sync_copy(red1, sh.at[pl.ds(sid * NPAD, NPAD)])
        pltpu.sync_copy(red2, sh.at[pl.ds((16 + sid) * NPAD, NPAD)])
        plsc.subcore_barrier()
        pltpu.sync_copy(sh, mrg)

        pbase = sid * _ROWS_PER_W
        for k in range(3):
            off = pbase + k * 16
            f = sent16
            s = sent16
            for t in range(16):
                c1 = mrg[pl.ds(t * NPAD + off, 16)]
                c2 = mrg[pl.ds((16 + t) * NPAD + off, 16)]
                s = jnp.minimum(jnp.maximum(f, c1), jnp.minimum(s, c2))
                f = jnp.minimum(f, c1)
            my = jnp.where(cid == 0, f, s)
            idx_v[pl.ds(k * 16, 16)] = jnp.minimum(my, B - 1)
            valid_v[pl.ds(k * 16, 16)] = (s < I_SENT).astype(jnp.int32)

        pltpu.async_copy(emb_hbm.at[idx_v], rows_v, sem).wait()
        rowbase = jnp.where(cid == 0, pbase, NPAIR + pbase)
        pltpu.sync_copy(rows_v.at[pl.ds(0, _ROWS_PER_W)],
                        rows_hbm.at[pl.ds(rowbase, _ROWS_PER_W)])

        @pl.when(cid == 0)
        def _():
            pltpu.sync_copy(valid_v.at[pl.ds(0, _ROWS_PER_W)],
                            valid2_hbm.at[pl.ds(pbase, _ROWS_PER_W)])

    return _g


def _sc_mine_gather(emb, p):
    return _make_sc_mine_gather()(emb, p)


def _dist_body(emb_ref, p_ref, a_ref, pos_ref, valid2_ref, loss_ref,
               minacc, qi16, qs8):
    step = pl.program_id(0)

    @pl.when(step == 0)
    def _():
        minacc[...] = jnp.full((16, NPAIR), F_BIG, jnp.bfloat16)
        q = lax.broadcasted_iota(jnp.int32, (DROWS, NPAIR), 1)
        qi16[...] = q.astype(jnp.int16)
        qs8[...] = (q // N_CLASSES).astype(jnp.int8)

    E = emb_ref[...]
    Eb = E.astype(jnp.bfloat16)
    Ab = a_ref[...].astype(jnp.bfloat16)
    G = lax.dot_general(Eb, Ab, (((1,), (1,)), ((), ())),
                        preferred_element_type=jnp.float32)
    en = jnp.sum(E * E, axis=1).astype(jnp.bfloat16)
    val = en[:, None] - 2.0 * G.astype(jnp.bfloat16)
    pr = p_ref[0, 0, :]
    pr16 = pr.astype(jnp.int16)[:, None]
    ps8 = (pr // N_CLASSES).astype(jnp.int8)[:, None]
    mask = (ps8 == qs8[...]) & (pr16 != qi16[...])
    masked = jnp.where(mask, val, jnp.bfloat16(F_BIG))
    mstep = jnp.min(masked.reshape(DROWS // 16, 16, NPAIR), axis=0)
    minacc[...] = jnp.minimum(minacc[...], mstep)

    @pl.when(step == DNBLK - 1)
    def _():
        A = a_ref[...]
        P = pos_ref[...]
        an2 = jnp.sum(A * A, axis=1)
        mn = jnp.min(minacc[...], axis=0).astype(jnp.float32)
        d_an = jnp.sqrt(jnp.maximum(mn + an2, 0.0))
        dif = A - P + EPS
        d_ap = jnp.sqrt(jnp.sum(dif * dif, axis=1))
        valid = (valid2_ref[...] > 0) & (mn < 1e29)
        term = jnp.where(valid, jnp.maximum(d_ap - d_an + MARGIN, 0.0), 0.0)
        total = jnp.sum(term)
        count = jnp.sum(valid.astype(jnp.float32))
        loss = jnp.where(count > 0.0, total / jnp.maximum(count, 1.0),
                         jnp.float32(0.0))
        loss_ref[...] = jnp.broadcast_to(loss, (1, 1))


def _dist(emb, p3, a_rows, p_rows, valid2):
    return pl.pallas_call(
        _dist_body,
        grid=(DNBLK,),
        in_specs=[
            pl.BlockSpec((DROWS, D), lambda s: (s, 0)),
            pl.BlockSpec((1, 1, DROWS), lambda s: (s, 0, 0)),
            pl.BlockSpec((NPAIR, D), lambda s: (0, 0)),
            pl.BlockSpec((NPAIR, D), lambda s: (1, 0)),
            pl.BlockSpec((NPAIR,), lambda s: (0,)),
        ],
        out_specs=pl.BlockSpec((1, 1), lambda s: (0, 0)),
        out_shape=jax.ShapeDtypeStruct((1, 1), jnp.float32),
        scratch_shapes=[pltpu.VMEM((16, NPAIR), jnp.bfloat16),
                        pltpu.VMEM((DROWS, NPAIR), jnp.int16),
                        pltpu.VMEM((DROWS, NPAIR), jnp.int8)],
    )(emb, p3, a_rows, p_rows, valid2)


def kernel(emb, labels, sbj):
    p = sbj * N_CLASSES + labels
    rows, valid2 = _sc_mine_gather(emb, p)
    loss = _dist(emb, p.reshape(DNBLK, 1, DROWS), rows, rows, valid2)
    return loss.reshape(())


# speedup vs baseline: 1.0669x; 1.0235x over previous
import functools

import jax
import jax.numpy as jnp
from jax import lax
from jax.experimental import pallas as pl
from jax.experimental.pallas import tpu as pltpu
from jax.experimental.pallas import tpu_sc as plsc

B = 16384
D = 256
N_CLASSES = 6
N_SUBJECTS = 100
NPAIR = 640
MARGIN = 1.0
EPS = 1e-6
ROWS = 1024
NBLK = B // ROWS
DROWS = 2048
DNBLK = B // DROWS
I_SENT = 2**30
F_BIG = 1e30


def _mine_body(p_ref, idx_ref, valid2_ref):
    qi = lax.broadcasted_iota(jnp.int32, (ROWS, NPAIR), 1)
    jg = lax.broadcasted_iota(jnp.int32, (ROWS, NPAIR), 0)
    f = jnp.full((NPAIR,), I_SENT, jnp.int32)
    s = jnp.full((NPAIR,), I_SENT, jnp.int32)
    for r in range(NBLK):
        pr = p_ref[r, :][:, None]
        m = jnp.where(pr == qi, jg, I_SENT)
        c1 = jnp.min(m, axis=0)
        c2 = jnp.min(jnp.where(m == c1[None, :], I_SENT, m), axis=0)
        c1 = jnp.where(c1 < I_SENT, c1 + r * ROWS, I_SENT)
        c2 = jnp.where(c2 < I_SENT, c2 + r * ROWS, I_SENT)
        s = jnp.minimum(jnp.maximum(f, c1), jnp.minimum(s, c2))
        f = jnp.minimum(f, c1)
    idx_ref[0:NPAIR] = jnp.minimum(f, B - 1)
    idx_ref[NPAIR:2 * NPAIR] = jnp.minimum(s, B - 1)
    valid2_ref[...] = (s < I_SENT).astype(jnp.int32)


def _mine(p2d):
    return pl.pallas_call(
        _mine_body,
        out_shape=(
            jax.ShapeDtypeStruct((2 * NPAIR,), jnp.int32),
            jax.ShapeDtypeStruct((NPAIR,), jnp.int32),
        ),
    )(p2d)


_ROWS_PER_W = (2 * NPAIR) // 32
NPAD = 656
SAMP_PER_SC = B // 16
SENTP = 2**20


@functools.cache
def _make_sc_mine_gather():
    mesh = plsc.VectorSubcoreMesh(core_axis_name="c", subcore_axis_name="s")

    @functools.partial(
        pl.kernel,
        mesh=mesh,
        compiler_params=pltpu.CompilerParams(needs_layout_passes=False),
        out_type=(
            jax.ShapeDtypeStruct((2 * NPAIR, D), jnp.float32),
            jax.ShapeDtypeStruct((NPAIR,), jnp.int32),
        ),
        scratch_types=[
            pltpu.VMEM((SAMP_PER_SC,), jnp.int32),
            pltpu.VMEM((16 * NPAD,), jnp.int32),
            pltpu.VMEM((16 * NPAD,), jnp.int32),
            pltpu.VMEM((NPAD,), jnp.int32),
            pltpu.VMEM((NPAD,), jnp.int32),
            pltpu.VMEM((2 * 16 * NPAD,), jnp.int32),
            pltpu.VMEM((48,), jnp.int32),
            pltpu.VMEM((48,), jnp.int32),
            pltpu.VMEM((_ROWS_PER_W, D), jnp.float32),
            pltpu.VMEM_SHARED((2 * 16 * NPAD,), jnp.int32),
            pltpu.SemaphoreType.DMA,
        ],
    )
    def _g(emb_hbm, p_hbm, rows_hbm, valid2_hbm, pchunk, loc1, loc2, red1,
           red2, mrg, idx_v, valid_v, rows_v, sh, sem):
        cid = lax.axis_index("c")
        sid = lax.axis_index("s")
        lane = lax.broadcasted_iota(jnp.int32, (16,), 0)
        sent16 = jnp.full((16,), I_SENT, jnp.int32)

        nchunk = NPAD // 16

        def init_body(i, c):
            for u in range(8):
                loc1[pl.ds((i * 8 + u) * 16, 16)] = sent16
                loc2[pl.ds((i * 8 + u) * 16, 16)] = sent16
            return c
        lax.fori_loop(0, 16 * nchunk // 8, init_body, 0)

        pltpu.sync_copy(p_hbm.at[pl.ds(sid * SAMP_PER_SC, SAMP_PER_SC)],
                        pchunk)

        lane_off = lane * NPAD

        def scan_body(k, c):
            for u in range(4):
                kk = k * 4 + u
                pv = pchunk[pl.ds(kk * 16, 16)]
                jv = sid * SAMP_PER_SC + kk * 16 + lane
                fidx = lane_off + pv
                cur1 = plsc.load_gather(loc1, [fidx])
                cur2 = plsc.load_gather(loc2, [fidx])
                plsc.store_scatter(loc1, [fidx], jnp.minimum(cur1, jv))
                plsc.store_scatter(loc2, [fidx],
                                   jnp.minimum(cur2, jnp.maximum(cur1, jv)))
            return c
        lax.fori_loop(0, SAMP_PER_SC // 64, scan_body, 0)

        def red_body(i, c):
            off = i * 16
            f = sent16
            s = sent16
            for l in range(16):
                c1 = loc1[pl.ds(l * NPAD + off, 16)]
                c2 = loc2[pl.ds(l * NPAD + off, 16)]
                s = jnp.minimum(jnp.maximum(f, c1), jnp.minimum(s, c2))
                f = jnp.minimum(f, c1)
            red1[pl.ds(off, 16)] = f
            red2[pl.ds(off, 16)] = s
            return c
        lax.fori_loop(0, nchunk, red_body, 0)

        pltpu.sync_copy(red1, sh.at[pl.ds(sid * NPAD, NPAD)])
        pltpu.sync_copy(red2, sh.at[pl.ds((16 + sid) * NPAD, NPAD)])
        plsc.subcore_barrier()
        pltpu.sync_copy(sh, mrg)

        pbase = sid * _ROWS_PER_W
        for k in range(3):
            off = pbase + k * 16
            f = sent16
            s = sent16
            for t in range(16):
                c1 = mrg[pl.ds(t * NPAD + off, 16)]
                c2 = mrg[pl.ds((16 + t) * NPAD + off, 16)]
                s = jnp.minimum(jnp.maximum(f, c1), jnp.minimum(s, c2))
                f = jnp.minimum(f, c1)
            my = jnp.where(cid == 0, f, s)
            idx_v[pl.ds(k * 16, 16)] = jnp.minimum(my, B - 1)
            valid_v[pl.ds(k * 16, 16)] = (s < I_SENT).astype(jnp.int32)

        pltpu.async_copy(emb_hbm.at[idx_v.at[pl.ds(0, _ROWS_PER_W)]],
                         rows_v, sem).wait()
        rowbase = jnp.where(cid == 0, pbase, NPAIR + pbase)
        pltpu.sync_copy(rows_v, rows_hbm.at[pl.ds(rowbase, _ROWS_PER_W)])

        @pl.when(cid == 0)
        def _():
            pltpu.sync_copy(valid_v.at[pl.ds(0, _ROWS_PER_W)],
                            valid2_hbm.at[pl.ds(pbase, _ROWS_PER_W)])

    return _g


def _sc_mine_gather(emb, p):
    return _make_sc_mine_gather()(emb, p)


def _dist_body(emb_ref, p_ref, a_ref, pos_ref, valid2_ref, loss_ref,
               minacc, qi16, qs8):
    step = pl.program_id(0)

    @pl.when(step == 0)
    def _():
        minacc[...] = jnp.full((16, NPAIR), F_BIG, jnp.bfloat16)
        q = lax.broadcasted_iota(jnp.int32, (DROWS, NPAIR), 1)
        qi16[...] = q.astype(jnp.int16)
        qs8[...] = (q // N_CLASSES).astype(jnp.int8)

    E = emb_ref[...]
    Eb = E.astype(jnp.bfloat16)
    Ab = a_ref[...].astype(jnp.bfloat16)
    G = lax.dot_general(Eb, Ab, (((1,), (1,)), ((), ())),
                        preferred_element_type=jnp.float32)
    en = jnp.sum(E * E, axis=1).astype(jnp.bfloat16)
    val = en[:, None] - 2.0 * G.astype(jnp.bfloat16)
    pr = p_ref[0, 0, :]
    pr16 = pr.astype(jnp.int16)[:, None]
    ps8 = (pr // N_CLASSES).astype(jnp.int8)[:, None]
    mask = (ps8 == qs8[...]) & (pr16 != qi16[...])
    masked = jnp.where(mask, val, jnp.bfloat16(F_BIG))
    mstep = jnp.min(masked.reshape(DROWS // 16, 16, NPAIR), axis=0)
    minacc[...] = jnp.minimum(minacc[...], mstep)

    @pl.when(step == DNBLK - 1)
    def _():
        A = a_ref[...]
        P = pos_ref[...]
        an2 = jnp.sum(A * A, axis=1)
        mn = jnp.min(minacc[...], axis=0).astype(jnp.float32)
        d_an = jnp.sqrt(jnp.maximum(mn + an2, 0.0))
        dif = A - P + EPS
        d_ap = jnp.sqrt(jnp.sum(dif * dif, axis=1))
        valid = (valid2_ref[...] > 0) & (mn < 1e29)
        term = jnp.where(valid, jnp.maximum(d_ap - d_an + MARGIN, 0.0), 0.0)
        total = jnp.sum(term)
        count = jnp.sum(valid.astype(jnp.float32))
        loss = jnp.where(count > 0.0, total / jnp.maximum(count, 1.0),
                         jnp.float32(0.0))
        loss_ref[...] = jnp.broadcast_to(loss, (1, 1))


def _dist(emb, p3, a_rows, p_rows, valid2):
    return pl.pallas_call(
        _dist_body,
        grid=(DNBLK,),
        in_specs=[
            pl.BlockSpec((DROWS, D), lambda s: (s, 0)),
            pl.BlockSpec((1, 1, DROWS), lambda s: (s, 0, 0)),
            pl.BlockSpec((NPAIR, D), lambda s: (0, 0)),
            pl.BlockSpec((NPAIR, D), lambda s: (1, 0)),
            pl.BlockSpec((NPAIR,), lambda s: (0,)),
        ],
        out_specs=pl.BlockSpec((1, 1), lambda s: (0, 0)),
        out_shape=jax.ShapeDtypeStruct((1, 1), jnp.float32),
        scratch_shapes=[pltpu.VMEM((16, NPAIR), jnp.bfloat16),
                        pltpu.VMEM((DROWS, NPAIR), jnp.int16),
                        pltpu.VMEM((DROWS, NPAIR), jnp.int8)],
    )(emb, p3, a_rows, p_rows, valid2)


def kernel(emb, labels, sbj):
    p = sbj * N_CLASSES + labels
    rows, valid2 = _sc_mine_gather(emb, p)
    loss = _dist(emb, p.reshape(DNBLK, 1, DROWS), rows, rows, valid2)
    return loss.reshape(())
